# baseline (speedup 1.0000x reference)
import functools

import numpy as np
import jax
import jax.numpy as jnp
from jax import lax
from jax.experimental import pallas as pl
from jax.experimental.pallas import tpu as pltpu
from jax.experimental.pallas import tpu_sc as plsc

R_MAX = 5.0
NUM_BESSEL = 8
P_CUT = 6.0
N_CHAN = 64
AVG_NEIGH = 16.0
N_NODES = 10000
N_EDGES = 160000
LDX = np.array([0, 1, 1, 1, 2, 2, 2, 2, 2])

E_BLK = 1280
N_BLK = 2000

_INTERPRET = False


def _edge_body(vec_ref, wr10, wr20, wr30, wr11, wr21, wr31,
               sh_ref, rw0_ref, rw1_ref):
    v = vec_ref[...]
    x, y, z = v[0:1], v[1:2], v[2:3]
    r2 = x * x + y * y + z * z + 1e-9
    r = jnp.sqrt(r2)
    inv = 1.0 / r
    ux, uy, uz = x * inv, y * inv, z * inv
    s3, s5, s15 = 3.0 ** 0.5, 5.0 ** 0.5, 15.0 ** 0.5
    B = v.shape[1]
    sh_rows = jnp.concatenate([
        jnp.ones_like(ux),
        s3 * ux, s3 * uy, s3 * uz,
        s15 * ux * uy, s15 * uy * uz, (s5 / 2.0) * (3.0 * uz * uz - 1.0),
        s15 * ux * uz, (s15 / 2.0) * (ux * ux - uy * uy),
        jnp.zeros((7, B), jnp.float32),
    ], axis=0)
    sh_ref[...] = sh_rows.T

    n = (jnp.arange(1, NUM_BESSEL + 1, dtype=jnp.int32)
         .astype(jnp.float32)[:, None])
    rb = (2.0 / R_MAX) ** 0.5 * jnp.sin(n * (jnp.pi / R_MAX) * r) / (r + 1e-9)
    p = P_CUT
    xx = r * (1.0 / R_MAX)
    x2 = xx * xx
    x4 = x2 * x2
    x6 = x4 * x2
    x7 = x6 * xx
    x8 = x7 * xx
    env = 1.0 - (p + 1.0) * (p + 2.0) / 2.0 * x6 + p * (p + 2.0) * x7 - p * (p + 1.0) / 2.0 * x8
    env = jnp.where(xx < 1.0, env, 0.0)
    ef = (rb * env).T

    def silu(t):
        return t / (1.0 + jnp.exp(-t))

    def mlp(w1, w2, w3, out_ref):
        h1 = silu(jax.lax.dot(ef, w1[...], preferred_element_type=jnp.float32))
        h2 = silu(jax.lax.dot(h1, w2[...], preferred_element_type=jnp.float32))
        rw = jax.lax.dot(h2, w3[...], preferred_element_type=jnp.float32)
        out_ref[0] = rw[:, 0:64]
        out_ref[1] = rw[:, 64:128]
        out_ref[2] = rw[:, 128:192]

    mlp(wr10, wr20, wr30, rw0_ref)
    mlp(wr11, wr21, wr31, rw1_ref)


def _edge_stage(vec_t, wr1, wr2, wr3p):
    nblk = N_EDGES // E_BLK
    full = lambda shape: pl.BlockSpec(shape, lambda i: (0,) * len(shape))
    return pl.pallas_call(
        _edge_body,
        grid=(nblk,),
        in_specs=[
            pl.BlockSpec((16, E_BLK), lambda i: (0, i)),
            full((NUM_BESSEL, 64)), full((64, 64)), full((64, 192)),
            full((NUM_BESSEL, 64)), full((64, 64)), full((64, 192)),
        ],
        out_specs=[
            pl.BlockSpec((E_BLK, 16), lambda i: (i, 0)),
            pl.BlockSpec((3, E_BLK, 64), lambda i: (0, i, 0)),
            pl.BlockSpec((3, E_BLK, 64), lambda i: (0, i, 0)),
        ],
        out_shape=[
            jax.ShapeDtypeStruct((N_EDGES, 16), jnp.float32),
            jax.ShapeDtypeStruct((3, N_EDGES, 64), jnp.float32),
            jax.ShapeDtypeStruct((3, N_EDGES, 64), jnp.float32),
        ],
        interpret=_INTERPRET,
    )(vec_t, wr1[0], wr2[0], wr3p[0], wr1[1], wr2[1], wr3p[1])


def _mm_body(x_ref, w_ref, o_ref):
    o_ref[...] = jax.lax.dot(x_ref[...], w_ref[...],
                             preferred_element_type=jnp.float32)


def _mm_stage(x, w):
    nblk = x.shape[0] // N_BLK
    return pl.pallas_call(
        _mm_body,
        grid=(nblk,),
        in_specs=[
            pl.BlockSpec((N_BLK, x.shape[1]), lambda i: (i, 0)),
            pl.BlockSpec(w.shape, lambda i: (0, 0)),
        ],
        out_specs=pl.BlockSpec((N_BLK, w.shape[1]), lambda i: (i, 0)),
        out_shape=jax.ShapeDtypeStruct((x.shape[0], w.shape[1]), jnp.float32),
        interpret=_INTERPRET,
    )(x, w)


def _node_body(first, A_ref, attrs_ref, nf_ref, wm_ref, wmix_ref, wc_ref,
               wmsg_ref, wskip_ref, wupn_ref, out_ref, h_ref):
    attrs = attrs_ref[...]
    anew = []
    for m in range(9):
        am = jax.lax.dot(A_ref[m], wm_ref[m], preferred_element_type=jnp.float32)
        anew.append(am)
    if first:
        anew[0] = anew[0] + jax.lax.dot(attrs, wmix_ref[...],
                                        preferred_element_type=jnp.float32)
    b1 = anew[0]
    b2 = anew[0] * anew[0]
    for m in range(1, 9):
        b2 = b2 + anew[m] * anew[m]
    b3 = b2 * b1
    wts = jax.lax.dot(attrs, wc_ref[...], preferred_element_type=jnp.float32)
    mm = wts[:, 0:64] * b1 + wts[:, 64:128] * b2 + wts[:, 128:192] * b3
    out = jax.lax.dot(mm, wmsg_ref[...], preferred_element_type=jnp.float32)
    if not first:
        out = out + jax.lax.dot(nf_ref[...], wskip_ref[...],
                                preferred_element_type=jnp.float32)
    out_ref[...] = out
    if first:
        h_ref[...] = jax.lax.dot(out, wupn_ref[...],
                                 preferred_element_type=jnp.float32)


def _node_stage(first, A, attrs, nf, wm, wmix, wc, wmsg, wskip, wupn):
    nblk = N_NODES // N_BLK
    full = lambda shape: pl.BlockSpec(shape, lambda i: (0,) * len(shape))
    out_specs = [pl.BlockSpec((N_BLK, 64), lambda i: (i, 0))]
    out_shape = [jax.ShapeDtypeStruct((N_NODES, 64), jnp.float32)]
    if first:
        out_specs.append(pl.BlockSpec((N_BLK, 64), lambda i: (i, 0)))
        out_shape.append(jax.ShapeDtypeStruct((N_NODES, 64), jnp.float32))
    else:
        out_specs.append(pl.BlockSpec((8, 64), lambda i: (0, 0)))
        out_shape.append(jax.ShapeDtypeStruct((8, 64), jnp.float32))
    res = pl.pallas_call(
        functools.partial(_node_body, first),
        grid=(nblk,),
        in_specs=[
            pl.BlockSpec((9, N_BLK, 64), lambda i: (0, i, 0)),
            pl.BlockSpec((N_BLK, 128), lambda i: (i, 0)),
            pl.BlockSpec((N_BLK, 64), lambda i: (i, 0)),
            full((9, 64, 64)), full((128, 64)), full((128, 192)),
            full((64, 64)), full((64, 64)), full((64, 64)),
        ],
        out_specs=out_specs,
        out_shape=out_shape,
        interpret=_INTERPRET,
    )(A, attrs, nf, wm, wmix, wc, wmsg, wskip, wupn)
    return res


def _sc_vec(coords_pad, src, dst):
    return coords_pad[dst] - coords_pad[src]


def _sc_scatter(rw, sh, h, src, dst):
    wlm = rw[LDX]
    msg = wlm * h[src][None, :, :] * sh.T[0:9, :, None]
    a = jnp.zeros((9, N_NODES, 64), jnp.float32).at[:, dst].add(msg)
    return a


def kernel(coordinates, node_attrs, edge_index, W_embed, W_up, Wr1, Wr2, Wr3,
           W_int, W_mix, Wc, W_msg, W_skip):
    src = edge_index[0].astype(jnp.int32)
    dst = edge_index[1].astype(jnp.int32)
    coords_pad = jnp.pad(coordinates, ((0, 0), (0, 13)))

    ldx = jnp.asarray(LDX)
    W_eff0 = W_embed @ W_up[0]
    Wr3p = Wr3.reshape(2, 64, N_CHAN, 3).transpose(0, 1, 3, 2).reshape(2, 64, 192)
    Wm = W_int[:, ldx] / AVG_NEIGH
    Wc_p = Wc.reshape(2, 128, N_CHAN, 3).transpose(0, 1, 3, 2).reshape(2, 128, 192)

    vec = _sc_vec(coords_pad, src, dst)
    vec_t = vec.T

    sh, rw0, rw1 = _edge_stage(vec_t, Wr1, Wr2, Wr3p)

    h0 = _mm_stage(node_attrs, W_eff0)

    dummy_nf = jnp.zeros((N_NODES, 64), jnp.float32)

    A0 = _sc_scatter(rw0, sh, h0, src, dst)
    out0, h1 = _node_stage(True, A0, node_attrs, dummy_nf,
                           Wm[0], W_mix, Wc_p[0], W_msg[0], W_skip[0], W_up[1])

    A1 = _sc_scatter(rw1, sh, h1, src, dst)
    out1, _ = _node_stage(False, A1, node_attrs, out0,
                          Wm[1], W_mix, Wc_p[1], W_msg[1], W_skip[1], W_up[1])

    return jnp.stack([out0, out1], axis=0)


# speedup vs baseline: 10.7665x; 10.7665x over previous
import functools

import numpy as np
import jax
import jax.numpy as jnp
from jax import lax
from jax.experimental import pallas as pl
from jax.experimental.pallas import tpu as pltpu
from jax.experimental.pallas import tpu_sc as plsc

R_MAX = 5.0
NUM_BESSEL = 8
P_CUT = 6.0
N_CHAN = 64
AVG_NEIGH = 16.0
N_NODES = 10000
N_EDGES = 160000
LDX = np.array([0, 1, 1, 1, 2, 2, 2, 2, 2])

E_BLK = 1280
N_BLK = 2000

_INTERPRET = False


def _edge_body(cs_ref, cd_ref, wr10, wr20, wr30, wr11, wr21, wr31,
               sh_ref, rw0_ref, rw1_ref):
    v = cd_ref[...] - cs_ref[...]
    x, y, z = v[0:1], v[1:2], v[2:3]
    r2 = x * x + y * y + z * z + 1e-9
    r = jnp.sqrt(r2)
    inv = 1.0 / r
    ux, uy, uz = x * inv, y * inv, z * inv
    s3, s5, s15 = 3.0 ** 0.5, 5.0 ** 0.5, 15.0 ** 0.5
    B = v.shape[1]
    sh_rows = jnp.concatenate([
        jnp.ones_like(ux),
        s3 * ux, s3 * uy, s3 * uz,
        s15 * ux * uy, s15 * uy * uz, (s5 / 2.0) * (3.0 * uz * uz - 1.0),
        s15 * ux * uz, (s15 / 2.0) * (ux * ux - uy * uy),
        jnp.zeros((7, B), jnp.float32),
    ], axis=0)
    sh_ref[...] = sh_rows.T

    n = (jnp.arange(1, NUM_BESSEL + 1, dtype=jnp.int32)
         .astype(jnp.float32)[:, None])
    rb = (2.0 / R_MAX) ** 0.5 * jnp.sin(n * (jnp.pi / R_MAX) * r) / (r + 1e-9)
    p = P_CUT
    xx = r * (1.0 / R_MAX)
    x2 = xx * xx
    x4 = x2 * x2
    x6 = x4 * x2
    x7 = x6 * xx
    x8 = x7 * xx
    env = 1.0 - (p + 1.0) * (p + 2.0) / 2.0 * x6 + p * (p + 2.0) * x7 - p * (p + 1.0) / 2.0 * x8
    env = jnp.where(xx < 1.0, env, 0.0)
    ef = (rb * env).T

    def silu(t):
        return t / (1.0 + jnp.exp(-t))

    def mlp(w1, w2, w3, out_ref):
        h1 = silu(jax.lax.dot(ef, w1[...], preferred_element_type=jnp.float32))
        h2 = silu(jax.lax.dot(h1, w2[...], preferred_element_type=jnp.float32))
        rw = jax.lax.dot(h2, w3[...], preferred_element_type=jnp.float32)
        out_ref[0] = rw[:, 0:64]
        out_ref[1] = rw[:, 64:128]
        out_ref[2] = rw[:, 128:192]

    mlp(wr10, wr20, wr30, rw0_ref)
    mlp(wr11, wr21, wr31, rw1_ref)


def _edge_stage(cs_t, cd_t, wr1, wr2, wr3p):
    nblk = N_EDGES // E_BLK
    full = lambda shape: pl.BlockSpec(shape, lambda i: (0,) * len(shape))
    return pl.pallas_call(
        _edge_body,
        grid=(nblk,),
        in_specs=[
            pl.BlockSpec((16, E_BLK), lambda i: (0, i)),
            pl.BlockSpec((16, E_BLK), lambda i: (0, i)),
            full((NUM_BESSEL, 64)), full((64, 64)), full((64, 192)),
            full((NUM_BESSEL, 64)), full((64, 64)), full((64, 192)),
        ],
        out_specs=[
            pl.BlockSpec((E_BLK, 16), lambda i: (i, 0)),
            pl.BlockSpec((3, E_BLK, 64), lambda i: (0, i, 0)),
            pl.BlockSpec((3, E_BLK, 64), lambda i: (0, i, 0)),
        ],
        out_shape=[
            jax.ShapeDtypeStruct((N_EDGES, 16), jnp.float32),
            jax.ShapeDtypeStruct((3, N_EDGES, 64), jnp.float32),
            jax.ShapeDtypeStruct((3, N_EDGES, 64), jnp.float32),
        ],
        interpret=_INTERPRET,
    )(cs_t, cd_t, wr1[0], wr2[0], wr3p[0], wr1[1], wr2[1], wr3p[1])


def _mm_body(x_ref, w_ref, o_ref):
    o_ref[...] = jax.lax.dot(x_ref[...], w_ref[...],
                             preferred_element_type=jnp.float32)


def _mm_stage(x, w):
    nblk = x.shape[0] // N_BLK
    return pl.pallas_call(
        _mm_body,
        grid=(nblk,),
        in_specs=[
            pl.BlockSpec((N_BLK, x.shape[1]), lambda i: (i, 0)),
            pl.BlockSpec(w.shape, lambda i: (0, 0)),
        ],
        out_specs=pl.BlockSpec((N_BLK, w.shape[1]), lambda i: (i, 0)),
        out_shape=jax.ShapeDtypeStruct((x.shape[0], w.shape[1]), jnp.float32),
        interpret=_INTERPRET,
    )(x, w)


def _node_body(first, A_ref, attrs_ref, nf_ref, wm_ref, wmix_ref, wc_ref,
               wmsg_ref, wskip_ref, wupn_ref, out_ref, h_ref):
    attrs = attrs_ref[...]
    anew = []
    for m in range(9):
        am = jax.lax.dot(A_ref[m], wm_ref[m], preferred_element_type=jnp.float32)
        anew.append(am)
    if first:
        anew[0] = anew[0] + jax.lax.dot(attrs, wmix_ref[...],
                                        preferred_element_type=jnp.float32)
    b1 = anew[0]
    b2 = anew[0] * anew[0]
    for m in range(1, 9):
        b2 = b2 + anew[m] * anew[m]
    b3 = b2 * b1
    wts = jax.lax.dot(attrs, wc_ref[...], preferred_element_type=jnp.float32)
    mm = wts[:, 0:64] * b1 + wts[:, 64:128] * b2 + wts[:, 128:192] * b3
    out = jax.lax.dot(mm, wmsg_ref[...], preferred_element_type=jnp.float32)
    if not first:
        out = out + jax.lax.dot(nf_ref[...], wskip_ref[...],
                                preferred_element_type=jnp.float32)
    out_ref[...] = out
    if first:
        h_ref[...] = jax.lax.dot(out, wupn_ref[...],
                                 preferred_element_type=jnp.float32)


def _node_stage(first, A, attrs, nf, wm, wmix, wc, wmsg, wskip, wupn):
    nblk = N_NODES // N_BLK
    full = lambda shape: pl.BlockSpec(shape, lambda i: (0,) * len(shape))
    out_specs = [pl.BlockSpec((N_BLK, 64), lambda i: (i, 0))]
    out_shape = [jax.ShapeDtypeStruct((N_NODES, 64), jnp.float32)]
    if first:
        out_specs.append(pl.BlockSpec((N_BLK, 64), lambda i: (i, 0)))
        out_shape.append(jax.ShapeDtypeStruct((N_NODES, 64), jnp.float32))
    else:
        out_specs.append(pl.BlockSpec((8, 64), lambda i: (0, 0)))
        out_shape.append(jax.ShapeDtypeStruct((8, 64), jnp.float32))
    res = pl.pallas_call(
        functools.partial(_node_body, first),
        grid=(nblk,),
        in_specs=[
            pl.BlockSpec((9, N_BLK, 64), lambda i: (0, i, 0)),
            pl.BlockSpec((N_BLK, 128), lambda i: (i, 0)),
            pl.BlockSpec((N_BLK, 64), lambda i: (i, 0)),
            full((9, 64, 64)), full((128, 64)), full((128, 192)),
            full((64, 64)), full((64, 64)), full((64, 64)),
        ],
        out_specs=out_specs,
        out_shape=out_shape,
        interpret=_INTERPRET,
    )(A, attrs, nf, wm, wmix, wc, wmsg, wskip, wupn)
    return res


_NW = 32
EV_W = N_EDGES // _NW


def _sc_mesh():
    return plsc.VectorSubcoreMesh(core_axis_name="c", subcore_axis_name="s")


def _vec_body(coords_hbm, src_hbm, dst_hbm, cs_hbm, cd_hbm, idx_v, rows_v, sem):
    wid = lax.axis_index("s") * 2 + lax.axis_index("c")
    base = wid * EV_W
    pltpu.sync_copy(src_hbm.at[pl.ds(base, EV_W)], idx_v)
    pltpu.async_copy(coords_hbm.at[idx_v], rows_v, sem).wait()
    pltpu.sync_copy(rows_v, cs_hbm.at[pl.ds(base, EV_W)])
    pltpu.sync_copy(dst_hbm.at[pl.ds(base, EV_W)], idx_v)
    pltpu.async_copy(coords_hbm.at[idx_v], rows_v, sem).wait()
    pltpu.sync_copy(rows_v, cd_hbm.at[pl.ds(base, EV_W)])


def _sc_vec_stage(coords_pad, src, dst):
    k = pl.kernel(
        _vec_body,
        out_type=[jax.ShapeDtypeStruct((N_EDGES, 16), jnp.float32),
                  jax.ShapeDtypeStruct((N_EDGES, 16), jnp.float32)],
        mesh=_sc_mesh(),
        scratch_types=[pltpu.VMEM((EV_W,), jnp.int32),
                       pltpu.VMEM((EV_W, 16), jnp.float32),
                       pltpu.SemaphoreType.DMA],
        compiler_params=pltpu.CompilerParams(use_tc_tiling_on_sc=False),
    )
    return k(coords_pad, src, dst)


_NT = 16
ES_W = N_EDGES // _NT
S_CHUNK = 80
S_NCHUNK = ES_W // S_CHUNK


def _scatter_body(sh_hbm, src_hbm, dst_hbm, rw_hbm, hq0, hq1, hq2, hq3,
                  zeros_hbm, out_hbm,
                  table, idx_v, dst_v, h_v, rw_v, sh_v, msg_v, sem):
    core = lax.axis_index("c")
    sid = lax.axis_index("s")
    base = sid * ES_W
    hqs = (hq0, hq1, hq2, hq3)

    def run_pass(q, hq_hbm):
        @pl.when(sid == 0)
        def _():
            pltpu.sync_copy(zeros_hbm, table)
        plsc.subcore_barrier()

        def chunk(k, carry):
            e0 = base + k * S_CHUNK
            pltpu.sync_copy(src_hbm.at[pl.ds(e0, S_CHUNK)], idx_v)
            pltpu.sync_copy(dst_hbm.at[pl.ds(e0, S_CHUNK)], dst_v)
            cp = pltpu.async_copy(hq_hbm.at[idx_v], h_v, sem)
            pltpu.sync_copy(sh_hbm.at[pl.ds(e0, S_CHUNK), :], sh_v)
            for l in range(3):
                pltpu.sync_copy(
                    rw_hbm.at[l, pl.ds(e0, S_CHUNK), pl.ds(16 * q, 16)],
                    rw_v.at[l])
            cp.wait()

            def edge(e, ecarry):
                h = h_v[e]
                shvec = sh_v[e]
                g0 = rw_v[0, e] * h
                g1 = rw_v[1, e] * h
                g2 = rw_v[2, e] * h
                msg_v[e, pl.ds(0, 16)] = g0 * shvec[0]
                for m in range(1, 4):
                    msg_v[e, pl.ds(16 * m, 16)] = g1 * shvec[m]
                for m in range(4, 9):
                    msg_v[e, pl.ds(16 * m, 16)] = g2 * shvec[m]
                return ecarry

            lax.fori_loop(0, S_CHUNK, edge, 0)
            pltpu.sync_copy(msg_v, table.at[dst_v], add=True)
            return carry

        lax.fori_loop(0, S_NCHUNK, chunk, 0)
        plsc.subcore_barrier()
        @pl.when(sid == 0)
        def _():
            pltpu.sync_copy(table, out_hbm.at[q])
        plsc.subcore_barrier()

    for p in range(2):
        for cval in range(2):
            q = 2 * p + cval
            @pl.when(core == cval)
            def _(q=q):
                run_pass(q, hqs[q])


def _sc_scatter_stage(rw, sh, hq, src, dst, zeros):
    k = pl.kernel(
        _scatter_body,
        out_type=jax.ShapeDtypeStruct((4, N_NODES, 144), jnp.float32),
        mesh=_sc_mesh(),
        scratch_types=[
            pltpu.VMEM_SHARED((N_NODES, 144), jnp.float32),
            pltpu.VMEM((S_CHUNK,), jnp.int32),
            pltpu.VMEM((S_CHUNK,), jnp.int32),
            pltpu.VMEM((S_CHUNK, 16), jnp.float32),
            pltpu.VMEM((3, S_CHUNK, 16), jnp.float32),
            pltpu.VMEM((S_CHUNK, 16), jnp.float32),
            pltpu.VMEM((S_CHUNK, 144), jnp.float32),
            pltpu.SemaphoreType.DMA,
        ],
        compiler_params=pltpu.CompilerParams(use_tc_tiling_on_sc=False),
    )
    return k(sh, src, dst, rw, hq[0], hq[1], hq[2], hq[3], zeros)


def _sc_scatter(rw, sh, h, src, dst, zeros):
    hq = [h[:, 16 * q:16 * (q + 1)] for q in range(4)]
    araw = _sc_scatter_stage(rw, sh, hq, src, dst, zeros)
    A = araw.reshape(4, N_NODES, 9, 16).transpose(2, 1, 0, 3).reshape(9, N_NODES, 64)
    return A


def kernel(coordinates, node_attrs, edge_index, W_embed, W_up, Wr1, Wr2, Wr3,
           W_int, W_mix, Wc, W_msg, W_skip):
    src = edge_index[0].astype(jnp.int32)
    dst = edge_index[1].astype(jnp.int32)
    coords_pad = jnp.pad(coordinates, ((0, 0), (0, 13)))

    ldx = jnp.asarray(LDX)
    W_eff0 = W_embed @ W_up[0]
    Wr3p = Wr3.reshape(2, 64, N_CHAN, 3).transpose(0, 1, 3, 2).reshape(2, 64, 192)
    Wm = W_int[:, ldx] / AVG_NEIGH
    Wc_p = Wc.reshape(2, 128, N_CHAN, 3).transpose(0, 1, 3, 2).reshape(2, 128, 192)

    cs, cd = _sc_vec_stage(coords_pad, src, dst)
    zeros = jnp.zeros((N_NODES, 144), jnp.float32)

    sh, rw0, rw1 = _edge_stage(cs.T, cd.T, Wr1, Wr2, Wr3p)

    h0 = _mm_stage(node_attrs, W_eff0)

    dummy_nf = jnp.zeros((N_NODES, 64), jnp.float32)

    A0 = _sc_scatter(rw0, sh, h0, src, dst, zeros)
    out0, h1 = _node_stage(True, A0, node_attrs, dummy_nf,
                           Wm[0], W_mix, Wc_p[0], W_msg[0], W_skip[0], W_up[1])

    A1 = _sc_scatter(rw1, sh, h1, src, dst, zeros)
    out1, _ = _node_stage(False, A1, node_attrs, out0,
                          Wm[1], W_mix, Wc_p[1], W_msg[1], W_skip[1], W_up[1])

    return jnp.stack([out0, out1], axis=0)


# speedup vs baseline: 11.5657x; 1.0742x over previous
import functools

import numpy as np
import jax
import jax.numpy as jnp
from jax import lax
from jax.experimental import pallas as pl
from jax.experimental.pallas import tpu as pltpu
from jax.experimental.pallas import tpu_sc as plsc

R_MAX = 5.0
NUM_BESSEL = 8
P_CUT = 6.0
N_CHAN = 64
AVG_NEIGH = 16.0
N_NODES = 10000
N_EDGES = 160000
LDX = np.array([0, 1, 1, 1, 2, 2, 2, 2, 2])

E_BLK = 1280
N_BLK = 2000

_INTERPRET = False


def _edge_body(vec_ref, wr10, wr20, wr30, wr11, wr21, wr31,
               sh_ref, rw0_ref, rw1_ref):
    v = vec_ref[...].T
    x, y, z = v[0:1], v[1:2], v[2:3]
    r2 = x * x + y * y + z * z + 1e-9
    r = jnp.sqrt(r2)
    inv = 1.0 / r
    ux, uy, uz = x * inv, y * inv, z * inv
    s3, s5, s15 = 3.0 ** 0.5, 5.0 ** 0.5, 15.0 ** 0.5
    B = v.shape[1]
    sh_rows = jnp.concatenate([
        jnp.ones_like(ux),
        s3 * ux, s3 * uy, s3 * uz,
        s15 * ux * uy, s15 * uy * uz, (s5 / 2.0) * (3.0 * uz * uz - 1.0),
        s15 * ux * uz, (s15 / 2.0) * (ux * ux - uy * uy),
        jnp.zeros((7, B), jnp.float32),
    ], axis=0)
    sh_ref[...] = sh_rows.T

    n = (jnp.arange(1, NUM_BESSEL + 1, dtype=jnp.int32)
         .astype(jnp.float32)[:, None])
    rb = (2.0 / R_MAX) ** 0.5 * jnp.sin(n * (jnp.pi / R_MAX) * r) / (r + 1e-9)
    p = P_CUT
    xx = r * (1.0 / R_MAX)
    x2 = xx * xx
    x4 = x2 * x2
    x6 = x4 * x2
    x7 = x6 * xx
    x8 = x7 * xx
    env = 1.0 - (p + 1.0) * (p + 2.0) / 2.0 * x6 + p * (p + 2.0) * x7 - p * (p + 1.0) / 2.0 * x8
    env = jnp.where(xx < 1.0, env, 0.0)
    ef = (rb * env).T

    def silu(t):
        return t / (1.0 + jnp.exp(-t))

    def mlp(w1, w2, w3, out_ref):
        h1 = silu(jax.lax.dot(ef, w1[...], preferred_element_type=jnp.float32))
        h2 = silu(jax.lax.dot(h1, w2[...], preferred_element_type=jnp.float32))
        rw = jax.lax.dot(h2, w3[...], preferred_element_type=jnp.float32)
        out_ref[0] = rw[:, 0:64]
        out_ref[1] = rw[:, 64:128]
        out_ref[2] = rw[:, 128:192]

    mlp(wr10, wr20, wr30, rw0_ref)
    mlp(wr11, wr21, wr31, rw1_ref)


def _edge_stage(vec, wr1, wr2, wr3p):
    nblk = N_EDGES // E_BLK
    full = lambda shape: pl.BlockSpec(shape, lambda i: (0,) * len(shape))
    return pl.pallas_call(
        _edge_body,
        grid=(nblk,),
        in_specs=[
            pl.BlockSpec((E_BLK, 16), lambda i: (i, 0)),
            full((NUM_BESSEL, 64)), full((64, 64)), full((64, 192)),
            full((NUM_BESSEL, 64)), full((64, 64)), full((64, 192)),
        ],
        out_specs=[
            pl.BlockSpec((E_BLK, 16), lambda i: (i, 0)),
            pl.BlockSpec((3, E_BLK, 64), lambda i: (0, i, 0)),
            pl.BlockSpec((3, E_BLK, 64), lambda i: (0, i, 0)),
        ],
        out_shape=[
            jax.ShapeDtypeStruct((N_EDGES, 16), jnp.float32),
            jax.ShapeDtypeStruct((3, N_EDGES, 64), jnp.float32),
            jax.ShapeDtypeStruct((3, N_EDGES, 64), jnp.float32),
        ],
        interpret=_INTERPRET,
    )(vec, wr1[0], wr2[0], wr3p[0], wr1[1], wr2[1], wr3p[1])


def _mm_body(x_ref, w_ref, o_ref):
    o_ref[...] = jax.lax.dot(x_ref[...], w_ref[...],
                             preferred_element_type=jnp.float32)


def _mm_stage(x, w):
    nblk = x.shape[0] // N_BLK
    return pl.pallas_call(
        _mm_body,
        grid=(nblk,),
        in_specs=[
            pl.BlockSpec((N_BLK, x.shape[1]), lambda i: (i, 0)),
            pl.BlockSpec(w.shape, lambda i: (0, 0)),
        ],
        out_specs=pl.BlockSpec((N_BLK, w.shape[1]), lambda i: (i, 0)),
        out_shape=jax.ShapeDtypeStruct((x.shape[0], w.shape[1]), jnp.float32),
        interpret=_INTERPRET,
    )(x, w)


def _node_body(first, A_ref, attrs_ref, nf_ref, wm_ref, wmix_ref, wc_ref,
               wmsg_ref, wskip_ref, wupn_ref, out_ref, h_ref):
    attrs = attrs_ref[...]
    anew = []
    for m in range(9):
        am = jax.lax.dot(A_ref[m], wm_ref[m], preferred_element_type=jnp.float32)
        anew.append(am)
    if first:
        anew[0] = anew[0] + jax.lax.dot(attrs, wmix_ref[...],
                                        preferred_element_type=jnp.float32)
    b1 = anew[0]
    b2 = anew[0] * anew[0]
    for m in range(1, 9):
        b2 = b2 + anew[m] * anew[m]
    b3 = b2 * b1
    wts = jax.lax.dot(attrs, wc_ref[...], preferred_element_type=jnp.float32)
    mm = wts[:, 0:64] * b1 + wts[:, 64:128] * b2 + wts[:, 128:192] * b3
    out = jax.lax.dot(mm, wmsg_ref[...], preferred_element_type=jnp.float32)
    if not first:
        out = out + jax.lax.dot(nf_ref[...], wskip_ref[...],
                                preferred_element_type=jnp.float32)
    out_ref[...] = out
    if first:
        h_ref[...] = jax.lax.dot(out, wupn_ref[...],
                                 preferred_element_type=jnp.float32)


def _node_stage(first, A, attrs, nf, wm, wmix, wc, wmsg, wskip, wupn):
    nblk = N_NODES // N_BLK
    full = lambda shape: pl.BlockSpec(shape, lambda i: (0,) * len(shape))
    out_specs = [pl.BlockSpec((N_BLK, 64), lambda i: (i, 0))]
    out_shape = [jax.ShapeDtypeStruct((N_NODES, 64), jnp.float32)]
    if first:
        out_specs.append(pl.BlockSpec((N_BLK, 64), lambda i: (i, 0)))
        out_shape.append(jax.ShapeDtypeStruct((N_NODES, 64), jnp.float32))
    else:
        out_specs.append(pl.BlockSpec((8, 64), lambda i: (0, 0)))
        out_shape.append(jax.ShapeDtypeStruct((8, 64), jnp.float32))
    res = pl.pallas_call(
        functools.partial(_node_body, first),
        grid=(nblk,),
        in_specs=[
            pl.BlockSpec((9, N_BLK, 64), lambda i: (0, i, 0)),
            pl.BlockSpec((N_BLK, 128), lambda i: (i, 0)),
            pl.BlockSpec((N_BLK, 64), lambda i: (i, 0)),
            full((9, 64, 64)), full((128, 64)), full((128, 192)),
            full((64, 64)), full((64, 64)), full((64, 64)),
        ],
        out_specs=out_specs,
        out_shape=out_shape,
        interpret=_INTERPRET,
    )(A, attrs, nf, wm, wmix, wc, wmsg, wskip, wupn)
    return res


_NW = 32
EV_W = N_EDGES // _NW


def _sc_mesh():
    return plsc.VectorSubcoreMesh(core_axis_name="c", subcore_axis_name="s")


_EV_CH = 1000


def _vec_body(coords_hbm, src_hbm, dst_hbm, vec_hbm, idx_v, cs_v, cd_v, sem):
    wid = lax.axis_index("s") * 2 + lax.axis_index("c")
    base = wid * EV_W

    def chunk(k, carry):
        e0 = base + k * _EV_CH
        pltpu.sync_copy(src_hbm.at[pl.ds(e0, _EV_CH)], idx_v)
        pltpu.async_copy(coords_hbm.at[idx_v], cs_v, sem).wait()
        pltpu.sync_copy(dst_hbm.at[pl.ds(e0, _EV_CH)], idx_v)
        pltpu.async_copy(coords_hbm.at[idx_v], cd_v, sem).wait()

        @plsc.parallel_loop(0, _EV_CH, step=1, unroll=4)
        def _(e):
            cs_v[e] = cd_v[e] - cs_v[e]

        pltpu.sync_copy(cs_v, vec_hbm.at[pl.ds(e0, _EV_CH)])
        return carry

    lax.fori_loop(0, EV_W // _EV_CH, chunk, 0)


def _sc_vec_stage(coords_pad, src, dst):
    k = pl.kernel(
        _vec_body,
        out_type=jax.ShapeDtypeStruct((N_EDGES, 16), jnp.float32),
        mesh=_sc_mesh(),
        scratch_types=[pltpu.VMEM((_EV_CH,), jnp.int32),
                       pltpu.VMEM((_EV_CH, 16), jnp.float32),
                       pltpu.VMEM((_EV_CH, 16), jnp.float32),
                       pltpu.SemaphoreType.DMA],
        compiler_params=pltpu.CompilerParams(use_tc_tiling_on_sc=False),
    )
    return k(coords_pad, src, dst)


_NT = 16
ES_W = N_EDGES // _NT
S_CHUNK = 80
S_NCHUNK = ES_W // S_CHUNK


def _scatter_body(sh_hbm, src_hbm, dst_hbm, rw_hbm, hq0, hq1, hq2, hq3,
                  zeros_hbm, out_hbm,
                  table, idx_v, dst_v, h_v, rw_v, sh_v, msg_v, sem):
    core = lax.axis_index("c")
    sid = lax.axis_index("s")
    base = sid * ES_W
    hqs = (hq0, hq1, hq2, hq3)

    def run_pass(q, hq_hbm):
        @pl.when(sid == 0)
        def _():
            pltpu.sync_copy(zeros_hbm, table)
        plsc.subcore_barrier()

        def chunk(k, carry):
            e0 = base + k * S_CHUNK
            pltpu.sync_copy(src_hbm.at[pl.ds(e0, S_CHUNK)], idx_v)
            pltpu.sync_copy(dst_hbm.at[pl.ds(e0, S_CHUNK)], dst_v)
            cp = pltpu.async_copy(hq_hbm.at[idx_v], h_v, sem)
            pltpu.sync_copy(sh_hbm.at[pl.ds(e0, S_CHUNK), :], sh_v)
            for l in range(3):
                pltpu.sync_copy(
                    rw_hbm.at[l, pl.ds(e0, S_CHUNK), pl.ds(16 * q, 16)],
                    rw_v.at[l])
            cp.wait()

            @plsc.parallel_loop(0, S_CHUNK, step=1, unroll=4)
            def _(e):
                h = h_v[e]
                shvec = sh_v[e]
                g0 = rw_v[0, e] * h
                g1 = rw_v[1, e] * h
                g2 = rw_v[2, e] * h
                msg_v[e, pl.ds(0, 16)] = g0 * shvec[0]
                for m in range(1, 4):
                    msg_v[e, pl.ds(16 * m, 16)] = g1 * shvec[m]
                for m in range(4, 9):
                    msg_v[e, pl.ds(16 * m, 16)] = g2 * shvec[m]
            pltpu.sync_copy(msg_v, table.at[dst_v], add=True)
            return carry

        lax.fori_loop(0, S_NCHUNK, chunk, 0)
        plsc.subcore_barrier()
        @pl.when(sid == 0)
        def _():
            pltpu.sync_copy(table, out_hbm.at[q])
        plsc.subcore_barrier()

    for p in range(2):
        for cval in range(2):
            q = 2 * p + cval
            @pl.when(core == cval)
            def _(q=q):
                run_pass(q, hqs[q])


def _sc_scatter_stage(rw, sh, hq, src, dst, zeros):
    k = pl.kernel(
        _scatter_body,
        out_type=jax.ShapeDtypeStruct((4, N_NODES, 144), jnp.float32),
        mesh=_sc_mesh(),
        scratch_types=[
            pltpu.VMEM_SHARED((N_NODES, 144), jnp.float32),
            pltpu.VMEM((S_CHUNK,), jnp.int32),
            pltpu.VMEM((S_CHUNK,), jnp.int32),
            pltpu.VMEM((S_CHUNK, 16), jnp.float32),
            pltpu.VMEM((3, S_CHUNK, 16), jnp.float32),
            pltpu.VMEM((S_CHUNK, 16), jnp.float32),
            pltpu.VMEM((S_CHUNK, 144), jnp.float32),
            pltpu.SemaphoreType.DMA,
        ],
        compiler_params=pltpu.CompilerParams(use_tc_tiling_on_sc=False),
    )
    return k(sh, src, dst, rw, hq[0], hq[1], hq[2], hq[3], zeros)


def _sc_scatter(rw, sh, h, src, dst, zeros):
    hq = [h[:, 16 * q:16 * (q + 1)] for q in range(4)]
    araw = _sc_scatter_stage(rw, sh, hq, src, dst, zeros)
    A = araw.reshape(4, N_NODES, 9, 16).transpose(2, 1, 0, 3).reshape(9, N_NODES, 64)
    return A


def kernel(coordinates, node_attrs, edge_index, W_embed, W_up, Wr1, Wr2, Wr3,
           W_int, W_mix, Wc, W_msg, W_skip):
    src = edge_index[0].astype(jnp.int32)
    dst = edge_index[1].astype(jnp.int32)
    coords_pad = jnp.pad(coordinates, ((0, 0), (0, 13)))

    ldx = jnp.asarray(LDX)
    W_eff0 = W_embed @ W_up[0]
    Wr3p = Wr3.reshape(2, 64, N_CHAN, 3).transpose(0, 1, 3, 2).reshape(2, 64, 192)
    Wm = W_int[:, ldx] / AVG_NEIGH
    Wc_p = Wc.reshape(2, 128, N_CHAN, 3).transpose(0, 1, 3, 2).reshape(2, 128, 192)

    vec = _sc_vec_stage(coords_pad, src, dst)
    zeros = jnp.zeros((N_NODES, 144), jnp.float32)

    sh, rw0, rw1 = _edge_stage(vec, Wr1, Wr2, Wr3p)

    h0 = _mm_stage(node_attrs, W_eff0)

    dummy_nf = jnp.zeros((N_NODES, 64), jnp.float32)

    A0 = _sc_scatter(rw0, sh, h0, src, dst, zeros)
    out0, h1 = _node_stage(True, A0, node_attrs, dummy_nf,
                           Wm[0], W_mix, Wc_p[0], W_msg[0], W_skip[0], W_up[1])

    A1 = _sc_scatter(rw1, sh, h1, src, dst, zeros)
    out1, _ = _node_stage(False, A1, node_attrs, out0,
                          Wm[1], W_mix, Wc_p[1], W_msg[1], W_skip[1], W_up[1])

    return jnp.stack([out0, out1], axis=0)


# speedup vs baseline: 17.0939x; 1.4780x over previous
import functools

import numpy as np
import jax
import jax.numpy as jnp
from jax import lax
from jax.experimental import pallas as pl
from jax.experimental.pallas import tpu as pltpu
from jax.experimental.pallas import tpu_sc as plsc

R_MAX = 5.0
NUM_BESSEL = 8
P_CUT = 6.0
N_CHAN = 64
AVG_NEIGH = 16.0
N_NODES = 10000
N_EDGES = 160000

E_BLK = 1280
N_BLK = 2000

_INTERPRET = False
_SC_PARAMS = None


def _sc_mesh():
    return plsc.VectorSubcoreMesh(core_axis_name="c", subcore_axis_name="s")


def _sc_params():
    return pltpu.CompilerParams(use_tc_tiling_on_sc=False)


def _edge_body(vec_ref, wr10, wr20, wr30, wr11, wr21, wr31,
               sh_ref, rw0_ref, rw1_ref):
    v = vec_ref[...].T
    x, y, z = v[0:1], v[1:2], v[2:3]
    r2 = x * x + y * y + z * z + 1e-9
    r = jnp.sqrt(r2)
    inv = 1.0 / r
    ux, uy, uz = x * inv, y * inv, z * inv
    s3, s5, s15 = 3.0 ** 0.5, 5.0 ** 0.5, 15.0 ** 0.5
    B = v.shape[1]
    sh_rows = jnp.concatenate([
        jnp.ones_like(ux),
        s3 * ux, s3 * uy, s3 * uz,
        s15 * ux * uy, s15 * uy * uz, (s5 / 2.0) * (3.0 * uz * uz - 1.0),
        s15 * ux * uz, (s15 / 2.0) * (ux * ux - uy * uy),
        jnp.zeros((7, B), jnp.float32),
    ], axis=0)
    sh_ref[...] = sh_rows.T

    n = (jnp.arange(1, NUM_BESSEL + 1, dtype=jnp.int32)
         .astype(jnp.float32)[:, None])
    rb = (2.0 / R_MAX) ** 0.5 * jnp.sin(n * (jnp.pi / R_MAX) * r) / (r + 1e-9)
    p = P_CUT
    xx = r * (1.0 / R_MAX)
    x2 = xx * xx
    x4 = x2 * x2
    x6 = x4 * x2
    x7 = x6 * xx
    x8 = x7 * xx
    env = 1.0 - (p + 1.0) * (p + 2.0) / 2.0 * x6 + p * (p + 2.0) * x7 - p * (p + 1.0) / 2.0 * x8
    env = jnp.where(xx < 1.0, env, 0.0)
    ef = (rb * env).T

    def silu(t):
        return t / (1.0 + jnp.exp(-t))

    def mlp(w1, w2, w3, out_ref):
        h1 = silu(jax.lax.dot(ef, w1[...], preferred_element_type=jnp.float32))
        h2 = silu(jax.lax.dot(h1, w2[...], preferred_element_type=jnp.float32))
        rw = jax.lax.dot(h2, w3[...], preferred_element_type=jnp.float32)
        out_ref[0] = rw[:, 0:64]
        out_ref[1] = rw[:, 64:128]
        out_ref[2] = rw[:, 128:192]

    mlp(wr10, wr20, wr30, rw0_ref)
    mlp(wr11, wr21, wr31, rw1_ref)


def _edge_stage(vec, wr1, wr2, wr3p):
    nblk = N_EDGES // E_BLK
    full = lambda shape: pl.BlockSpec(shape, lambda i: (0,) * len(shape))
    return pl.pallas_call(
        _edge_body,
        grid=(nblk,),
        in_specs=[
            pl.BlockSpec((E_BLK, 16), lambda i: (i, 0)),
            full((NUM_BESSEL, 64)), full((64, 64)), full((64, 192)),
            full((NUM_BESSEL, 64)), full((64, 64)), full((64, 192)),
        ],
        out_specs=[
            pl.BlockSpec((E_BLK, 16), lambda i: (i, 0)),
            pl.BlockSpec((3, E_BLK, 64), lambda i: (0, i, 0)),
            pl.BlockSpec((3, E_BLK, 64), lambda i: (0, i, 0)),
        ],
        out_shape=[
            jax.ShapeDtypeStruct((N_EDGES, 16), jnp.float32),
            jax.ShapeDtypeStruct((3, N_EDGES, 64), jnp.float32),
            jax.ShapeDtypeStruct((3, N_EDGES, 64), jnp.float32),
        ],
        interpret=_INTERPRET,
    )(vec, wr1[0], wr2[0], wr3p[0], wr1[1], wr2[1], wr3p[1])


def _mm_body(x_ref, w_ref, o_ref):
    o_ref[...] = jax.lax.dot(x_ref[...], w_ref[...],
                             preferred_element_type=jnp.float32)


def _mm_stage(x, w):
    nblk = x.shape[0] // N_BLK
    return pl.pallas_call(
        _mm_body,
        grid=(nblk,),
        in_specs=[
            pl.BlockSpec((N_BLK, x.shape[1]), lambda i: (i, 0)),
            pl.BlockSpec(w.shape, lambda i: (0, 0)),
        ],
        out_specs=pl.BlockSpec((N_BLK, w.shape[1]), lambda i: (i, 0)),
        out_shape=jax.ShapeDtypeStruct((x.shape[0], w.shape[1]), jnp.float32),
        interpret=_INTERPRET,
    )(x, w)


def _gpack_body(hs_ref, rw_ref, sh_ref, g_ref):
    hs = hs_ref[...]
    for l in range(3):
        g_ref[:, l, :] = rw_ref[l] * hs
    sh = sh_ref[...]
    g_ref[:, 3, :] = jnp.concatenate([sh, sh, sh, sh], axis=1)


def _gpack_stage(hs, rw, sh):
    nblk = N_EDGES // E_BLK
    return pl.pallas_call(
        _gpack_body,
        grid=(nblk,),
        in_specs=[
            pl.BlockSpec((E_BLK, 64), lambda i: (i, 0)),
            pl.BlockSpec((3, E_BLK, 64), lambda i: (0, i, 0)),
            pl.BlockSpec((E_BLK, 16), lambda i: (i, 0)),
        ],
        out_specs=pl.BlockSpec((E_BLK, 4, 64), lambda i: (i, 0, 0)),
        out_shape=jax.ShapeDtypeStruct((N_EDGES, 4, 64), jnp.float32),
        interpret=_INTERPRET,
    )(hs, rw, sh)


def _node_body(first, r0, r1, r2, r3, attrs_ref, nf_ref, wq_ref, wmix_ref,
               wc_ref, wmsg_ref, wskip_ref, wupn_ref, out_ref, h_ref):
    attrs = attrs_ref[...]
    anew = jax.lax.dot(r0[...], wq_ref[0], preferred_element_type=jnp.float32)
    for q, rq in enumerate((r1, r2, r3)):
        anew = anew + jax.lax.dot(rq[...], wq_ref[q + 1],
                                  preferred_element_type=jnp.float32)
    a0 = anew[:, 0:64]
    if first:
        a0 = a0 + jax.lax.dot(attrs, wmix_ref[...],
                              preferred_element_type=jnp.float32)
    b2 = a0 * a0
    for m in range(1, 9):
        am = anew[:, 64 * m:64 * (m + 1)]
        b2 = b2 + am * am
    b1 = a0
    b3 = b2 * b1
    wts = jax.lax.dot(attrs, wc_ref[...], preferred_element_type=jnp.float32)
    mm = wts[:, 0:64] * b1 + wts[:, 64:128] * b2 + wts[:, 128:192] * b3
    out = jax.lax.dot(mm, wmsg_ref[...], preferred_element_type=jnp.float32)
    if not first:
        out = out + jax.lax.dot(nf_ref[...], wskip_ref[...],
                                preferred_element_type=jnp.float32)
    out_ref[...] = out
    if first:
        h_ref[...] = jax.lax.dot(out, wupn_ref[...],
                                 preferred_element_type=jnp.float32)


def _node_stage(first, araw, attrs, nf, wq, wmix, wc, wmsg, wskip, wupn):
    nblk = N_NODES // N_BLK
    full = lambda shape: pl.BlockSpec(shape, lambda i: (0,) * len(shape))
    out_specs = [pl.BlockSpec((N_BLK, 64), lambda i: (i, 0))]
    out_shape = [jax.ShapeDtypeStruct((N_NODES, 64), jnp.float32)]
    if first:
        out_specs.append(pl.BlockSpec((N_BLK, 64), lambda i: (i, 0)))
        out_shape.append(jax.ShapeDtypeStruct((N_NODES, 64), jnp.float32))
    else:
        out_specs.append(pl.BlockSpec((8, 64), lambda i: (0, 0)))
        out_shape.append(jax.ShapeDtypeStruct((8, 64), jnp.float32))
    qspec = pl.BlockSpec((N_BLK, 144), lambda i: (i, 0))
    return pl.pallas_call(
        functools.partial(_node_body, first),
        grid=(nblk,),
        in_specs=[
            qspec, qspec, qspec, qspec,
            pl.BlockSpec((N_BLK, 128), lambda i: (i, 0)),
            pl.BlockSpec((N_BLK, 64), lambda i: (i, 0)),
            full((4, 144, 576)), full((128, 64)), full((128, 192)),
            full((64, 64)), full((64, 64)), full((64, 64)),
        ],
        out_specs=out_specs,
        out_shape=out_shape,
        interpret=_INTERPRET,
    )(araw[0], araw[1], araw[2], araw[3], attrs, nf,
      wq, wmix, wc, wmsg, wskip, wupn)


_NW = 32
EV_W = N_EDGES // _NW
_EV_CH = 1000


def _vec_body(coords_hbm, src_hbm, dst_hbm, vec_hbm, idx_v, cs_v, cd_v, sem):
    wid = lax.axis_index("s") * 2 + lax.axis_index("c")
    base = wid * EV_W

    def chunk(k, carry):
        e0 = base + k * _EV_CH
        pltpu.sync_copy(src_hbm.at[pl.ds(e0, _EV_CH)], idx_v)
        pltpu.async_copy(coords_hbm.at[idx_v], cs_v, sem).wait()
        pltpu.sync_copy(dst_hbm.at[pl.ds(e0, _EV_CH)], idx_v)
        pltpu.async_copy(coords_hbm.at[idx_v], cd_v, sem).wait()

        @plsc.parallel_loop(0, _EV_CH, step=1, unroll=4)
        def _(e):
            cs_v[e] = cd_v[e] - cs_v[e]

        pltpu.sync_copy(cs_v, vec_hbm.at[pl.ds(e0, _EV_CH)])
        return carry

    lax.fori_loop(0, EV_W // _EV_CH, chunk, 0)


def _sc_vec_stage(coords_pad, src, dst):
    k = pl.kernel(
        _vec_body,
        out_type=jax.ShapeDtypeStruct((N_EDGES, 16), jnp.float32),
        mesh=_sc_mesh(),
        scratch_types=[pltpu.VMEM((_EV_CH,), jnp.int32),
                       pltpu.VMEM((_EV_CH, 16), jnp.float32),
                       pltpu.VMEM((_EV_CH, 16), jnp.float32),
                       pltpu.SemaphoreType.DMA],
        compiler_params=_sc_params(),
    )
    return k(coords_pad, src, dst)


def _hgather_body(h_hbm, src_hbm, hs_hbm, idx_v, rows_v, sem):
    wid = lax.axis_index("s") * 2 + lax.axis_index("c")
    base = wid * EV_W

    def chunk(k, carry):
        e0 = base + k * _EV_CH
        pltpu.sync_copy(src_hbm.at[pl.ds(e0, _EV_CH)], idx_v)
        pltpu.async_copy(h_hbm.at[idx_v], rows_v, sem).wait()
        pltpu.sync_copy(rows_v, hs_hbm.at[pl.ds(e0, _EV_CH)])
        return carry

    lax.fori_loop(0, EV_W // _EV_CH, chunk, 0)


def _sc_hgather_stage(h, src):
    k = pl.kernel(
        _hgather_body,
        out_type=jax.ShapeDtypeStruct((N_EDGES, 64), jnp.float32),
        mesh=_sc_mesh(),
        scratch_types=[pltpu.VMEM((_EV_CH,), jnp.int32),
                       pltpu.VMEM((_EV_CH, 64), jnp.float32),
                       pltpu.SemaphoreType.DMA],
        compiler_params=_sc_params(),
    )
    return k(h, src)


_NT = 16
ES_W = N_EDGES // _NT
S_CHUNK = 80
S_NCHUNK = ES_W // S_CHUNK
_UNROLL6 = 6
_S_MAIN = (S_NCHUNK // _UNROLL6) * _UNROLL6


def _scatter_body(dst_hbm, g_hbm, zeros_hbm, out_hbm,
                  table, dstb, dsc, gbuf, msg,
                  sin0, sin1, sin2, ssc0, ssc1):
    core = lax.axis_index("c")
    sid = lax.axis_index("s")
    base = sid * ES_W
    sins = (sin0, sin1, sin2)
    sscs = (ssc0, ssc1)

    def run_pass(q):
        @pl.when(sid == 0)
        def _():
            pltpu.sync_copy(zeros_hbm, table)
        plsc.subcore_barrier()

        def issue_in(t, b):
            e0 = base + t * S_CHUNK
            pltpu.async_copy(dst_hbm.at[pl.ds(e0, S_CHUNK)], dstb.at[b],
                             sins[b])
            pltpu.async_copy(
                g_hbm.at[pl.ds(e0, S_CHUNK), :, pl.ds(16 * q, 16)],
                gbuf.at[b], sins[b])

        def wait_in(t, b):
            e0 = base + t * S_CHUNK
            pltpu.make_async_copy(dst_hbm.at[pl.ds(e0, S_CHUNK)], dstb.at[b],
                                  sins[b]).wait()
            pltpu.make_async_copy(
                g_hbm.at[pl.ds(e0, S_CHUNK), :, pl.ds(16 * q, 16)],
                gbuf.at[b], sins[b]).wait()

        def wait_sc(mb):
            pltpu.make_async_copy(msg.at[mb], table.at[dsc.at[mb]],
                                  sscs[mb]).wait()

        def do_chunk(t, b, mb, guard_sc):
            wait_in(t, b)
            if guard_sc is None:
                wait_sc(mb)
            elif guard_sc:
                @pl.when(t >= 2)
                def _():
                    wait_sc(mb)
            for i in range(S_CHUNK // 16):
                dsc[mb, pl.ds(16 * i, 16)] = dstb[b, pl.ds(16 * i, 16)]

            @plsc.parallel_loop(0, S_CHUNK, step=1, unroll=4)
            def _(e):
                shv = gbuf[b, e, 3]
                g0 = gbuf[b, e, 0]
                g1 = gbuf[b, e, 1]
                g2 = gbuf[b, e, 2]
                msg[mb, e, pl.ds(0, 16)] = g0
                for m in range(1, 4):
                    msg[mb, e, pl.ds(16 * m, 16)] = g1 * shv[m]
                for m in range(4, 9):
                    msg[mb, e, pl.ds(16 * m, 16)] = g2 * shv[m]

            pltpu.async_copy(msg.at[mb], table.at[dsc.at[mb]], sscs[mb],
                             add=True)

        for t in range(3):
            issue_in(t, t)

        def six(i, carry):
            j = i * _UNROLL6
            for c in range(_UNROLL6):
                do_chunk(j + c, c % 3, c % 2, True)
                issue_in(j + c + 3, c % 3)
            return carry

        lax.fori_loop(0, _S_MAIN // _UNROLL6, six, 0)
        for t in range(_S_MAIN, S_NCHUNK):
            do_chunk(t, t % 3, t % 2, None)
            if t + 3 < S_NCHUNK:
                issue_in(t + 3, t % 3)
        wait_sc((S_NCHUNK - 2) % 2)
        wait_sc((S_NCHUNK - 1) % 2)

        plsc.subcore_barrier()
        @pl.when(sid == 0)
        def _():
            pltpu.sync_copy(table, out_hbm.at[q])
        plsc.subcore_barrier()

    for p in range(2):
        for cval in range(2):
            @pl.when(core == cval)
            def _(q=2 * p + cval):
                run_pass(q)


def _sc_scatter_stage(gdata, dst, zeros):
    k = pl.kernel(
        _scatter_body,
        out_type=jax.ShapeDtypeStruct((4, N_NODES, 144), jnp.float32),
        mesh=_sc_mesh(),
        scratch_types=[
            pltpu.VMEM_SHARED((N_NODES, 144), jnp.float32),
            pltpu.VMEM((3, S_CHUNK), jnp.int32),
            pltpu.VMEM((2, S_CHUNK), jnp.int32),
            pltpu.VMEM((3, S_CHUNK, 4, 16), jnp.float32),
            pltpu.VMEM((2, S_CHUNK, 144), jnp.float32),
            pltpu.SemaphoreType.DMA,
            pltpu.SemaphoreType.DMA,
            pltpu.SemaphoreType.DMA,
            pltpu.SemaphoreType.DMA,
            pltpu.SemaphoreType.DMA,
        ],
        compiler_params=_sc_params(),
    )
    return k(dst, gdata, zeros)


def _sc_scatter(rw, sh, h, src, dst, zeros):
    hs = _sc_hgather_stage(h, src)
    gdata = _gpack_stage(hs, rw, sh)
    araw = _sc_scatter_stage(gdata, dst, zeros)
    return araw


def kernel(coordinates, node_attrs, edge_index, W_embed, W_up, Wr1, Wr2, Wr3,
           W_int, W_mix, Wc, W_msg, W_skip):
    src = edge_index[0].astype(jnp.int32)
    dst = edge_index[1].astype(jnp.int32)
    coords_pad = jnp.pad(coordinates, ((0, 0), (0, 13)))

    ldx = jnp.asarray(np.array([0, 1, 1, 1, 2, 2, 2, 2, 2]))
    W_eff0 = W_embed @ W_up[0]
    Wr3p = Wr3.reshape(2, 64, N_CHAN, 3).transpose(0, 1, 3, 2).reshape(2, 64, 192)
    Wm = W_int[:, ldx] / AVG_NEIGH
    eye9 = jnp.eye(9, dtype=jnp.float32)
    Wq = jnp.einsum('imqcd,mn->iqmcnd', Wm.reshape(2, 9, 4, 16, 64),
                    eye9).reshape(2, 4, 144, 576)
    Wc_p = Wc.reshape(2, 128, N_CHAN, 3).transpose(0, 1, 3, 2).reshape(2, 128, 192)

    vec = _sc_vec_stage(coords_pad, src, dst)
    zeros = jnp.zeros((N_NODES, 144), jnp.float32)

    sh, rw0, rw1 = _edge_stage(vec, Wr1, Wr2, Wr3p)

    h0 = _mm_stage(node_attrs, W_eff0)

    dummy_nf = jnp.zeros((N_NODES, 64), jnp.float32)

    A0 = _sc_scatter(rw0, sh, h0, src, dst, zeros)
    out0, h1 = _node_stage(True, A0, node_attrs, dummy_nf,
                           Wq[0], W_mix, Wc_p[0], W_msg[0], W_skip[0], W_up[1])

    A1 = _sc_scatter(rw1, sh, h1, src, dst, zeros)
    out1, _ = _node_stage(False, A1, node_attrs, out0,
                          Wq[1], W_mix, Wc_p[1], W_msg[1], W_skip[1], W_up[1])

    return jnp.stack([out0, out1], axis=0)


# speedup vs baseline: 19.9301x; 1.1659x over previous
import functools

import numpy as np
import jax
import jax.numpy as jnp
from jax import lax
from jax.experimental import pallas as pl
from jax.experimental.pallas import tpu as pltpu
from jax.experimental.pallas import tpu_sc as plsc

R_MAX = 5.0
NUM_BESSEL = 8
P_CUT = 6.0
N_CHAN = 64
AVG_NEIGH = 16.0
N_NODES = 10000
N_EDGES = 160000

E_BLK = 1280
N_BLK = 2000

_INTERPRET = False
_SC_PARAMS = None


def _sc_mesh():
    return plsc.VectorSubcoreMesh(core_axis_name="c", subcore_axis_name="s")


def _sc_params():
    return pltpu.CompilerParams(use_tc_tiling_on_sc=False)


def _edge_body(vec_ref, wr10, wr20, wr30, wr11, wr21, wr31,
               rw0_ref, rw1_ref):
    v = vec_ref[...].T
    x, y, z = v[0:1], v[1:2], v[2:3]
    r2 = x * x + y * y + z * z + 1e-9
    r = jnp.sqrt(r2)
    inv = 1.0 / r
    ux, uy, uz = x * inv, y * inv, z * inv
    s3, s5, s15 = 3.0 ** 0.5, 5.0 ** 0.5, 15.0 ** 0.5
    B = v.shape[1]
    sh_rows = jnp.concatenate([
        jnp.ones_like(ux),
        s3 * ux, s3 * uy, s3 * uz,
        s15 * ux * uy, s15 * uy * uz, (s5 / 2.0) * (3.0 * uz * uz - 1.0),
        s15 * ux * uz, (s15 / 2.0) * (ux * ux - uy * uy),
        jnp.zeros((7, B), jnp.float32),
    ], axis=0)

    n = (jnp.arange(1, NUM_BESSEL + 1, dtype=jnp.int32)
         .astype(jnp.float32)[:, None])
    rb = (2.0 / R_MAX) ** 0.5 * jnp.sin(n * (jnp.pi / R_MAX) * r) / (r + 1e-9)
    p = P_CUT
    xx = r * (1.0 / R_MAX)
    x2 = xx * xx
    x4 = x2 * x2
    x6 = x4 * x2
    x7 = x6 * xx
    x8 = x7 * xx
    env = 1.0 - (p + 1.0) * (p + 2.0) / 2.0 * x6 + p * (p + 2.0) * x7 - p * (p + 1.0) / 2.0 * x8
    env = jnp.where(xx < 1.0, env, 0.0)
    ef = (rb * env).T

    def silu(t):
        return t / (1.0 + jnp.exp(-t))

    shp = sh_rows.T
    sh4 = jnp.concatenate([shp, shp, shp, shp], axis=1)

    def mlp(w1, w2, w3, out_ref):
        h1 = silu(jax.lax.dot(ef, w1[...], preferred_element_type=jnp.float32))
        h2 = silu(jax.lax.dot(h1, w2[...], preferred_element_type=jnp.float32))
        rw = jax.lax.dot(h2, w3[...], preferred_element_type=jnp.float32)
        out_ref[:, 0, :] = rw[:, 0:64]
        out_ref[:, 1, :] = rw[:, 64:128]
        out_ref[:, 2, :] = rw[:, 128:192]
        out_ref[:, 3, :] = sh4

    mlp(wr10, wr20, wr30, rw0_ref)
    mlp(wr11, wr21, wr31, rw1_ref)


def _edge_stage(vec, wr1, wr2, wr3p):
    nblk = N_EDGES // E_BLK
    full = lambda shape: pl.BlockSpec(shape, lambda i: (0,) * len(shape))
    return pl.pallas_call(
        _edge_body,
        grid=(nblk,),
        in_specs=[
            pl.BlockSpec((E_BLK, 16), lambda i: (i, 0)),
            full((NUM_BESSEL, 64)), full((64, 64)), full((64, 192)),
            full((NUM_BESSEL, 64)), full((64, 64)), full((64, 192)),
        ],
        out_specs=[
            pl.BlockSpec((E_BLK, 4, 64), lambda i: (i, 0, 0)),
            pl.BlockSpec((E_BLK, 4, 64), lambda i: (i, 0, 0)),
        ],
        out_shape=[
            jax.ShapeDtypeStruct((N_EDGES, 4, 64), jnp.float32),
            jax.ShapeDtypeStruct((N_EDGES, 4, 64), jnp.float32),
        ],
        interpret=_INTERPRET,
    )(vec, wr1[0], wr2[0], wr3p[0], wr1[1], wr2[1], wr3p[1])


def _mm_body(x_ref, w_ref, o_ref):
    o_ref[...] = jax.lax.dot(x_ref[...], w_ref[...],
                             preferred_element_type=jnp.float32)


def _mm_stage(x, w):
    nblk = x.shape[0] // N_BLK
    return pl.pallas_call(
        _mm_body,
        grid=(nblk,),
        in_specs=[
            pl.BlockSpec((N_BLK, x.shape[1]), lambda i: (i, 0)),
            pl.BlockSpec(w.shape, lambda i: (0, 0)),
        ],
        out_specs=pl.BlockSpec((N_BLK, w.shape[1]), lambda i: (i, 0)),
        out_shape=jax.ShapeDtypeStruct((x.shape[0], w.shape[1]), jnp.float32),
        interpret=_INTERPRET,
    )(x, w)


def _node_body(first, r0, r1, r2, r3, attrs_ref, nf_ref, wq_ref, wmix_ref,
               wc_ref, wmsg_ref, wskip_ref, wupn_ref, out_ref, h_ref):
    attrs = attrs_ref[...]
    anew = jax.lax.dot(r0[...], wq_ref[0], preferred_element_type=jnp.float32)
    for q, rq in enumerate((r1, r2, r3)):
        anew = anew + jax.lax.dot(rq[...], wq_ref[q + 1],
                                  preferred_element_type=jnp.float32)
    a0 = anew[:, 0:64]
    if first:
        a0 = a0 + jax.lax.dot(attrs, wmix_ref[...],
                              preferred_element_type=jnp.float32)
    b2 = a0 * a0
    for m in range(1, 9):
        am = anew[:, 64 * m:64 * (m + 1)]
        b2 = b2 + am * am
    b1 = a0
    b3 = b2 * b1
    wts = jax.lax.dot(attrs, wc_ref[...], preferred_element_type=jnp.float32)
    mm = wts[:, 0:64] * b1 + wts[:, 64:128] * b2 + wts[:, 128:192] * b3
    out = jax.lax.dot(mm, wmsg_ref[...], preferred_element_type=jnp.float32)
    if not first:
        out = out + jax.lax.dot(nf_ref[...], wskip_ref[...],
                                preferred_element_type=jnp.float32)
    out_ref[...] = out
    if first:
        h_ref[...] = jax.lax.dot(out, wupn_ref[...],
                                 preferred_element_type=jnp.float32)


def _node_stage(first, araw, attrs, nf, wq, wmix, wc, wmsg, wskip, wupn):
    nblk = N_NODES // N_BLK
    full = lambda shape: pl.BlockSpec(shape, lambda i: (0,) * len(shape))
    out_specs = [pl.BlockSpec((N_BLK, 64), lambda i: (i, 0))]
    out_shape = [jax.ShapeDtypeStruct((N_NODES, 64), jnp.float32)]
    if first:
        out_specs.append(pl.BlockSpec((N_BLK, 64), lambda i: (i, 0)))
        out_shape.append(jax.ShapeDtypeStruct((N_NODES, 64), jnp.float32))
    else:
        out_specs.append(pl.BlockSpec((8, 64), lambda i: (0, 0)))
        out_shape.append(jax.ShapeDtypeStruct((8, 64), jnp.float32))
    qspec = pl.BlockSpec((N_BLK, 144), lambda i: (i, 0))
    return pl.pallas_call(
        functools.partial(_node_body, first),
        grid=(nblk,),
        in_specs=[
            qspec, qspec, qspec, qspec,
            pl.BlockSpec((N_BLK, 128), lambda i: (i, 0)),
            pl.BlockSpec((N_BLK, 64), lambda i: (i, 0)),
            full((4, 144, 576)), full((128, 64)), full((128, 192)),
            full((64, 64)), full((64, 64)), full((64, 64)),
        ],
        out_specs=out_specs,
        out_shape=out_shape,
        interpret=_INTERPRET,
    )(araw[0], araw[1], araw[2], araw[3], attrs, nf,
      wq, wmix, wc, wmsg, wskip, wupn)


_NW = 32
EV_W = N_EDGES // _NW
_EV_CH = 1000


def _vec_body(coords_hbm, src_hbm, dst_hbm, vec_hbm, idx_v, cs_v, cd_v, sem):
    wid = lax.axis_index("s") * 2 + lax.axis_index("c")
    base = wid * EV_W

    def chunk(k, carry):
        e0 = base + k * _EV_CH
        pltpu.sync_copy(src_hbm.at[pl.ds(e0, _EV_CH)], idx_v)
        pltpu.async_copy(coords_hbm.at[idx_v], cs_v, sem).wait()
        pltpu.sync_copy(dst_hbm.at[pl.ds(e0, _EV_CH)], idx_v)
        pltpu.async_copy(coords_hbm.at[idx_v], cd_v, sem).wait()

        @plsc.parallel_loop(0, _EV_CH, step=1, unroll=4)
        def _(e):
            cs_v[e] = cd_v[e] - cs_v[e]

        pltpu.sync_copy(cs_v, vec_hbm.at[pl.ds(e0, _EV_CH)])
        return carry

    lax.fori_loop(0, EV_W // _EV_CH, chunk, 0)


def _sc_vec_stage(coords_pad, src, dst):
    k = pl.kernel(
        _vec_body,
        out_type=jax.ShapeDtypeStruct((N_EDGES, 16), jnp.float32),
        mesh=_sc_mesh(),
        scratch_types=[pltpu.VMEM((_EV_CH,), jnp.int32),
                       pltpu.VMEM((_EV_CH, 16), jnp.float32),
                       pltpu.VMEM((_EV_CH, 16), jnp.float32),
                       pltpu.SemaphoreType.DMA],
        compiler_params=_sc_params(),
    )
    return k(coords_pad, src, dst)


def _hgather_body(h_hbm, src_hbm, hs_hbm, idx_v, rows_v, sem):
    wid = lax.axis_index("s") * 2 + lax.axis_index("c")
    base = wid * EV_W

    def chunk(k, carry):
        e0 = base + k * _EV_CH
        pltpu.sync_copy(src_hbm.at[pl.ds(e0, _EV_CH)], idx_v)
        pltpu.async_copy(h_hbm.at[idx_v], rows_v, sem).wait()
        pltpu.sync_copy(rows_v, hs_hbm.at[pl.ds(e0, _EV_CH)])
        return carry

    lax.fori_loop(0, EV_W // _EV_CH, chunk, 0)


def _sc_hgather_stage(h, src):
    k = pl.kernel(
        _hgather_body,
        out_type=jax.ShapeDtypeStruct((N_EDGES, 64), jnp.float32),
        mesh=_sc_mesh(),
        scratch_types=[pltpu.VMEM((_EV_CH,), jnp.int32),
                       pltpu.VMEM((_EV_CH, 64), jnp.float32),
                       pltpu.SemaphoreType.DMA],
        compiler_params=_sc_params(),
    )
    return k(h, src)


_NT = 16
ES_W = N_EDGES // _NT
S_CHUNK = 80
S_NCHUNK = ES_W // S_CHUNK
_S_MAIN = S_NCHUNK - 1


def _scatter_body(dst_hbm, g_hbm, hs_hbm, zeros_hbm, out_hbm,
                  table, dstb, dsc, gbuf, hbuf, msg,
                  sin0, sin1, ssc0, ssc1):
    core = lax.axis_index("c")
    sid = lax.axis_index("s")
    base = sid * ES_W
    sins = (sin0, sin1)
    sscs = (ssc0, ssc1)

    def run_pass(q):
        @pl.when(sid == 0)
        def _():
            pltpu.sync_copy(zeros_hbm, table)
        plsc.subcore_barrier()

        def in_copies(t, b):
            e0 = base + t * S_CHUNK
            return (
                pltpu.make_async_copy(dst_hbm.at[pl.ds(e0, S_CHUNK)],
                                      dstb.at[b], sins[b]),
                pltpu.make_async_copy(
                    g_hbm.at[pl.ds(e0, S_CHUNK), :, pl.ds(16 * q, 16)],
                    gbuf.at[b], sins[b]),
                pltpu.make_async_copy(
                    hs_hbm.at[pl.ds(e0, S_CHUNK), pl.ds(16 * q, 16)],
                    hbuf.at[b], sins[b]),
            )

        def issue_in(t, b):
            for cp in in_copies(t, b):
                cp.start()

        def wait_in(t, b):
            for cp in in_copies(t, b):
                cp.wait()

        def wait_sc(mb):
            pltpu.make_async_copy(msg.at[mb], table.at[dsc.at[mb]],
                                  sscs[mb]).wait()

        def do_chunk(t, b, mb, guard_sc, guard_issue):
            wait_in(t, b)
            if guard_sc:
                @pl.when(t >= 2)
                def _():
                    wait_sc(mb)
            else:
                wait_sc(mb)
            for i in range(S_CHUNK // 16):
                dsc[mb, pl.ds(16 * i, 16)] = dstb[b, pl.ds(16 * i, 16)]

            @plsc.parallel_loop(0, S_CHUNK, step=1, unroll=4)
            def _(e):
                shv = gbuf[b, e, 3]
                h = hbuf[b, e]
                g0 = gbuf[b, e, 0] * h
                g1 = gbuf[b, e, 1] * h
                g2 = gbuf[b, e, 2] * h
                msg[mb, e, pl.ds(0, 16)] = g0
                for m in range(1, 4):
                    msg[mb, e, pl.ds(16 * m, 16)] = g1 * shv[m]
                for m in range(4, 9):
                    msg[mb, e, pl.ds(16 * m, 16)] = g2 * shv[m]

            pltpu.async_copy(msg.at[mb], table.at[dsc.at[mb]], sscs[mb],
                             add=True)
            if guard_issue == "always":
                issue_in(t + 2, b)
            elif guard_issue == "when":
                @pl.when(t + 2 < S_NCHUNK)
                def _():
                    issue_in(t + 2, b)

        issue_in(0, 0)
        issue_in(1, 1)

        def pair(i, carry):
            j = i * 2
            do_chunk(j, 0, 0, True, "always")
            do_chunk(j + 1, 1, 1, True, "when")
            return carry

        lax.fori_loop(0, _S_MAIN // 2, pair, 0)
        do_chunk(S_NCHUNK - 1, (S_NCHUNK - 1) % 2, (S_NCHUNK - 1) % 2,
                 False, "none")
        wait_sc((S_NCHUNK - 2) % 2)
        wait_sc((S_NCHUNK - 1) % 2)

        plsc.subcore_barrier()
        @pl.when(sid == 0)
        def _():
            pltpu.sync_copy(table, out_hbm.at[q])
        plsc.subcore_barrier()

    for p in range(2):
        for cval in range(2):
            @pl.when(core == cval)
            def _(q=2 * p + cval):
                run_pass(q)


def _sc_scatter_stage(gdata, hs, dst, zeros):
    k = pl.kernel(
        _scatter_body,
        out_type=jax.ShapeDtypeStruct((4, N_NODES, 144), jnp.float32),
        mesh=_sc_mesh(),
        scratch_types=[
            pltpu.VMEM_SHARED((N_NODES, 144), jnp.float32),
            pltpu.VMEM((2, S_CHUNK), jnp.int32),
            pltpu.VMEM((2, S_CHUNK), jnp.int32),
            pltpu.VMEM((2, S_CHUNK, 4, 16), jnp.float32),
            pltpu.VMEM((2, S_CHUNK, 16), jnp.float32),
            pltpu.VMEM((2, S_CHUNK, 144), jnp.float32),
            pltpu.SemaphoreType.DMA,
            pltpu.SemaphoreType.DMA,
            pltpu.SemaphoreType.DMA,
            pltpu.SemaphoreType.DMA,
        ],
        compiler_params=_sc_params(),
    )
    return k(dst, gdata, hs, zeros)


def _sc_scatter(edata, h, src, dst, zeros):
    hs = _sc_hgather_stage(h, src)
    araw = _sc_scatter_stage(edata, hs, dst, zeros)
    return araw


def kernel(coordinates, node_attrs, edge_index, W_embed, W_up, Wr1, Wr2, Wr3,
           W_int, W_mix, Wc, W_msg, W_skip):
    src = edge_index[0].astype(jnp.int32)
    dst = edge_index[1].astype(jnp.int32)
    coords_pad = jnp.pad(coordinates, ((0, 0), (0, 13)))

    ldx = jnp.asarray(np.array([0, 1, 1, 1, 2, 2, 2, 2, 2]))
    W_eff0 = W_embed @ W_up[0]
    Wr3p = Wr3.reshape(2, 64, N_CHAN, 3).transpose(0, 1, 3, 2).reshape(2, 64, 192)
    Wm = W_int[:, ldx] / AVG_NEIGH
    eye9 = jnp.eye(9, dtype=jnp.float32)
    Wq = jnp.einsum('imqcd,mn->iqmcnd', Wm.reshape(2, 9, 4, 16, 64),
                    eye9).reshape(2, 4, 144, 576)
    Wc_p = Wc.reshape(2, 128, N_CHAN, 3).transpose(0, 1, 3, 2).reshape(2, 128, 192)

    vec = _sc_vec_stage(coords_pad, src, dst)
    zeros = jnp.zeros((N_NODES, 144), jnp.float32)

    ed0, ed1 = _edge_stage(vec, Wr1, Wr2, Wr3p)

    h0 = _mm_stage(node_attrs, W_eff0)

    dummy_nf = jnp.zeros((N_NODES, 64), jnp.float32)

    A0 = _sc_scatter(ed0, h0, src, dst, zeros)
    out0, h1 = _node_stage(True, A0, node_attrs, dummy_nf,
                           Wq[0], W_mix, Wc_p[0], W_msg[0], W_skip[0], W_up[1])

    A1 = _sc_scatter(ed1, h1, src, dst, zeros)
    out1, _ = _node_stage(False, A1, node_attrs, out0,
                          Wq[1], W_mix, Wc_p[1], W_msg[1], W_skip[1], W_up[1])

    return jnp.stack([out0, out1], axis=0)


# speedup vs baseline: 30.4964x; 1.5302x over previous
import functools

import numpy as np
import jax
import jax.numpy as jnp
from jax import lax
from jax.experimental import pallas as pl
from jax.experimental.pallas import tpu as pltpu
from jax.experimental.pallas import tpu_sc as plsc

R_MAX = 5.0
NUM_BESSEL = 8
P_CUT = 6.0
N_CHAN = 64
AVG_NEIGH = 16.0
N_NODES = 10000
N_EDGES = 160000

E_BLK = 6400
N_BLK = 2000

_INTERPRET = False
_SC_PARAMS = None


def _sc_mesh():
    return plsc.VectorSubcoreMesh(core_axis_name="c", subcore_axis_name="s")


def _sc_params():
    return pltpu.CompilerParams(use_tc_tiling_on_sc=False)


def _edge_body(vec_ref, wr10, wr20, wr30, wr11, wr21, wr31,
               sh_ref, rw0_ref, rw1_ref):
    v = vec_ref[...].T
    x, y, z = v[0:1], v[1:2], v[2:3]
    r2 = x * x + y * y + z * z + 1e-9
    r = jnp.sqrt(r2)
    inv = 1.0 / r
    ux, uy, uz = x * inv, y * inv, z * inv
    s3, s5, s15 = 3.0 ** 0.5, 5.0 ** 0.5, 15.0 ** 0.5
    B = v.shape[1]
    sh_rows = jnp.concatenate([
        jnp.ones_like(ux),
        s3 * ux, s3 * uy, s3 * uz,
        s15 * ux * uy, s15 * uy * uz, (s5 / 2.0) * (3.0 * uz * uz - 1.0),
        s15 * ux * uz, (s15 / 2.0) * (ux * ux - uy * uy),
        jnp.zeros((7, B), jnp.float32),
    ], axis=0)

    n = (jnp.arange(1, NUM_BESSEL + 1, dtype=jnp.int32)
         .astype(jnp.float32)[:, None])
    rb = (2.0 / R_MAX) ** 0.5 * jnp.sin(n * (jnp.pi / R_MAX) * r) / (r + 1e-9)
    p = P_CUT
    xx = r * (1.0 / R_MAX)
    x2 = xx * xx
    x4 = x2 * x2
    x6 = x4 * x2
    x7 = x6 * xx
    x8 = x7 * xx
    env = 1.0 - (p + 1.0) * (p + 2.0) / 2.0 * x6 + p * (p + 2.0) * x7 - p * (p + 1.0) / 2.0 * x8
    env = jnp.where(xx < 1.0, env, 0.0)
    ef = (rb * env).T

    def silu(t):
        return t / (1.0 + jnp.exp(-t))

    sh_ref[...] = sh_rows.T

    def mlp(w1, w2, w3, out_ref):
        h1 = silu(jax.lax.dot(ef, w1[...], preferred_element_type=jnp.float32))
        h2 = silu(jax.lax.dot(h1, w2[...], preferred_element_type=jnp.float32))
        out_ref[...] = jax.lax.dot(h2, w3[...],
                                   preferred_element_type=jnp.float32)

    mlp(wr10, wr20, wr30, rw0_ref)
    mlp(wr11, wr21, wr31, rw1_ref)


def _edge_stage(vec, wr1, wr2, wr3p):
    nblk = N_EDGES // E_BLK
    full = lambda shape: pl.BlockSpec(shape, lambda i: (0,) * len(shape))
    return pl.pallas_call(
        _edge_body,
        grid=(nblk,),
        in_specs=[
            pl.BlockSpec((E_BLK, 16), lambda i: (i, 0)),
            full((NUM_BESSEL, 64)), full((64, 64)), full((64, 192)),
            full((NUM_BESSEL, 64)), full((64, 64)), full((64, 192)),
        ],
        out_specs=[
            pl.BlockSpec((E_BLK, 16), lambda i: (i, 0)),
            pl.BlockSpec((E_BLK, 192), lambda i: (i, 0)),
            pl.BlockSpec((E_BLK, 192), lambda i: (i, 0)),
        ],
        out_shape=[
            jax.ShapeDtypeStruct((N_EDGES, 16), jnp.float32),
            jax.ShapeDtypeStruct((N_EDGES, 192), jnp.float32),
            jax.ShapeDtypeStruct((N_EDGES, 192), jnp.float32),
        ],
        interpret=_INTERPRET,
    )(vec, wr1[0], wr2[0], wr3p[0], wr1[1], wr2[1], wr3p[1])


def _mm_body(x_ref, w_ref, o_ref):
    o_ref[...] = jax.lax.dot(x_ref[...], w_ref[...],
                             preferred_element_type=jnp.float32)


def _mm_stage(x, w):
    nblk = x.shape[0] // N_BLK
    return pl.pallas_call(
        _mm_body,
        grid=(nblk,),
        in_specs=[
            pl.BlockSpec((N_BLK, x.shape[1]), lambda i: (i, 0)),
            pl.BlockSpec(w.shape, lambda i: (0, 0)),
        ],
        out_specs=pl.BlockSpec((N_BLK, w.shape[1]), lambda i: (i, 0)),
        out_shape=jax.ShapeDtypeStruct((x.shape[0], w.shape[1]), jnp.float32),
        interpret=_INTERPRET,
    )(x, w)


def _node_body(first, r0, r1, r2, r3, attrs_ref, nf_ref, wq_ref, wmix_ref,
               wc_ref, wmsg_ref, wskip_ref, wupn_ref, out_ref, h_ref):
    attrs = attrs_ref[...]
    anew = jax.lax.dot(r0[...], wq_ref[0], preferred_element_type=jnp.float32)
    for q, rq in enumerate((r1, r2, r3)):
        anew = anew + jax.lax.dot(rq[...], wq_ref[q + 1],
                                  preferred_element_type=jnp.float32)
    a0 = anew[:, 0:64]
    if first:
        a0 = a0 + jax.lax.dot(attrs, wmix_ref[...],
                              preferred_element_type=jnp.float32)
    b2 = a0 * a0
    for m in range(1, 9):
        am = anew[:, 64 * m:64 * (m + 1)]
        b2 = b2 + am * am
    b1 = a0
    b3 = b2 * b1
    wts = jax.lax.dot(attrs, wc_ref[...], preferred_element_type=jnp.float32)
    mm = wts[:, 0:64] * b1 + wts[:, 64:128] * b2 + wts[:, 128:192] * b3
    out = jax.lax.dot(mm, wmsg_ref[...], preferred_element_type=jnp.float32)
    if not first:
        out = out + jax.lax.dot(nf_ref[...], wskip_ref[...],
                                preferred_element_type=jnp.float32)
    out_ref[...] = out
    if first:
        h_ref[...] = jax.lax.dot(out, wupn_ref[...],
                                 preferred_element_type=jnp.float32)


def _node_stage(first, araw, attrs, nf, wq, wmix, wc, wmsg, wskip, wupn):
    nblk = N_NODES // N_BLK
    full = lambda shape: pl.BlockSpec(shape, lambda i: (0,) * len(shape))
    out_specs = [pl.BlockSpec((N_BLK, 64), lambda i: (i, 0))]
    out_shape = [jax.ShapeDtypeStruct((N_NODES, 64), jnp.float32)]
    if first:
        out_specs.append(pl.BlockSpec((N_BLK, 64), lambda i: (i, 0)))
        out_shape.append(jax.ShapeDtypeStruct((N_NODES, 64), jnp.float32))
    else:
        out_specs.append(pl.BlockSpec((8, 64), lambda i: (0, 0)))
        out_shape.append(jax.ShapeDtypeStruct((8, 64), jnp.float32))
    qspec = pl.BlockSpec((N_BLK, 144), lambda i: (i, 0))
    return pl.pallas_call(
        functools.partial(_node_body, first),
        grid=(nblk,),
        in_specs=[
            qspec, qspec, qspec, qspec,
            pl.BlockSpec((N_BLK, 128), lambda i: (i, 0)),
            pl.BlockSpec((N_BLK, 64), lambda i: (i, 0)),
            full((4, 144, 576)), full((128, 64)), full((128, 192)),
            full((64, 64)), full((64, 64)), full((64, 64)),
        ],
        out_specs=out_specs,
        out_shape=out_shape,
        interpret=_INTERPRET,
    )(araw[0], araw[1], araw[2], araw[3], attrs, nf,
      wq, wmix, wc, wmsg, wskip, wupn)


_NW = 32
EV_W = N_EDGES // _NW
_EV_CH = 1000


def _vec_body(coords_hbm, src_hbm, dst_hbm, vec_hbm, idx_v, cs_v, cd_v, sem):
    wid = lax.axis_index("s") * 2 + lax.axis_index("c")
    base = wid * EV_W

    def chunk(k, carry):
        e0 = base + k * _EV_CH
        pltpu.sync_copy(src_hbm.at[pl.ds(e0, _EV_CH)], idx_v)
        pltpu.async_copy(coords_hbm.at[idx_v], cs_v, sem).wait()
        pltpu.sync_copy(dst_hbm.at[pl.ds(e0, _EV_CH)], idx_v)
        pltpu.async_copy(coords_hbm.at[idx_v], cd_v, sem).wait()

        @plsc.parallel_loop(0, _EV_CH, step=1, unroll=4)
        def _(e):
            cs_v[e] = cd_v[e] - cs_v[e]

        pltpu.sync_copy(cs_v, vec_hbm.at[pl.ds(e0, _EV_CH)])
        return carry

    lax.fori_loop(0, EV_W // _EV_CH, chunk, 0)


def _sc_vec_stage(coords_pad, src, dst):
    k = pl.kernel(
        _vec_body,
        out_type=jax.ShapeDtypeStruct((N_EDGES, 16), jnp.float32),
        mesh=_sc_mesh(),
        scratch_types=[pltpu.VMEM((_EV_CH,), jnp.int32),
                       pltpu.VMEM((_EV_CH, 16), jnp.float32),
                       pltpu.VMEM((_EV_CH, 16), jnp.float32),
                       pltpu.SemaphoreType.DMA],
        compiler_params=_sc_params(),
    )
    return k(coords_pad, src, dst)


def _hgather_body(h_hbm, src_hbm, hs_hbm, idx_v, rows_v, sem):
    wid = lax.axis_index("s") * 2 + lax.axis_index("c")
    base = wid * EV_W

    def chunk(k, carry):
        e0 = base + k * _EV_CH
        pltpu.sync_copy(src_hbm.at[pl.ds(e0, _EV_CH)], idx_v)
        pltpu.async_copy(h_hbm.at[idx_v], rows_v, sem).wait()
        pltpu.sync_copy(rows_v, hs_hbm.at[pl.ds(e0, _EV_CH)])
        return carry

    lax.fori_loop(0, EV_W // _EV_CH, chunk, 0)


def _sc_hgather_stage(h, src):
    k = pl.kernel(
        _hgather_body,
        out_type=jax.ShapeDtypeStruct((N_EDGES, 64), jnp.float32),
        mesh=_sc_mesh(),
        scratch_types=[pltpu.VMEM((_EV_CH,), jnp.int32),
                       pltpu.VMEM((_EV_CH, 64), jnp.float32),
                       pltpu.SemaphoreType.DMA],
        compiler_params=_sc_params(),
    )
    return k(h, src)


_NT = 16
ES_W = N_EDGES // _NT
S_CHUNK = 80
S_NCHUNK = ES_W // S_CHUNK
_S_MAIN = S_NCHUNK - 1


def _scatter_body(dst_hbm, rw_hbm, sh_hbm, hs_hbm, zeros_hbm, out_hbm,
                  table, dstb, dsc, gbuf, shb, hbuf, msg,
                  sin0, sin1, ssc0, ssc1):
    core = lax.axis_index("c")
    sid = lax.axis_index("s")
    base = sid * ES_W
    sins = (sin0, sin1)
    sscs = (ssc0, ssc1)

    def run_pass(q):
        @pl.when(sid == 0)
        def _():
            pltpu.sync_copy(zeros_hbm, table)
        plsc.subcore_barrier()

        def in_copies(t, b):
            e0 = base + t * S_CHUNK
            cps = [
                pltpu.make_async_copy(dst_hbm.at[pl.ds(e0, S_CHUNK)],
                                      dstb.at[b], sins[b]),
                pltpu.make_async_copy(sh_hbm.at[pl.ds(e0, S_CHUNK), :],
                                      shb.at[b], sins[b]),
                pltpu.make_async_copy(
                    hs_hbm.at[pl.ds(e0, S_CHUNK), pl.ds(16 * q, 16)],
                    hbuf.at[b], sins[b]),
            ]
            for l in range(3):
                cps.append(pltpu.make_async_copy(
                    rw_hbm.at[pl.ds(e0, S_CHUNK), pl.ds(64 * l + 16 * q, 16)],
                    gbuf.at[b, l], sins[b]))
            return cps

        def issue_in(t, b):
            for cp in in_copies(t, b):
                cp.start()

        def wait_in(t, b):
            for cp in in_copies(t, b):
                cp.wait()

        def wait_sc(mb):
            pltpu.make_async_copy(msg.at[mb], table.at[dsc.at[mb]],
                                  sscs[mb]).wait()

        def do_chunk(t, b, mb, guard_sc, guard_issue):
            wait_in(t, b)
            if guard_sc:
                @pl.when(t >= 2)
                def _():
                    wait_sc(mb)
            else:
                wait_sc(mb)
            for i in range(S_CHUNK // 16):
                dsc[mb, pl.ds(16 * i, 16)] = dstb[b, pl.ds(16 * i, 16)]

            @plsc.parallel_loop(0, S_CHUNK, step=1, unroll=4)
            def _(e):
                shv = shb[b, e]
                h = hbuf[b, e]
                g0 = gbuf[b, 0, e] * h
                g1 = gbuf[b, 1, e] * h
                g2 = gbuf[b, 2, e] * h
                msg[mb, e, pl.ds(0, 16)] = g0
                for m in range(1, 4):
                    msg[mb, e, pl.ds(16 * m, 16)] = g1 * shv[m]
                for m in range(4, 9):
                    msg[mb, e, pl.ds(16 * m, 16)] = g2 * shv[m]

            pltpu.async_copy(msg.at[mb], table.at[dsc.at[mb]], sscs[mb],
                             add=True)
            if guard_issue == "always":
                issue_in(t + 2, b)
            elif guard_issue == "when":
                @pl.when(t + 2 < S_NCHUNK)
                def _():
                    issue_in(t + 2, b)

        issue_in(0, 0)
        issue_in(1, 1)

        def pair(i, carry):
            j = i * 2
            do_chunk(j, 0, 0, True, "always")
            do_chunk(j + 1, 1, 1, True, "when")
            return carry

        lax.fori_loop(0, _S_MAIN // 2, pair, 0)
        do_chunk(S_NCHUNK - 1, (S_NCHUNK - 1) % 2, (S_NCHUNK - 1) % 2,
                 False, "none")
        wait_sc((S_NCHUNK - 2) % 2)
        wait_sc((S_NCHUNK - 1) % 2)

        plsc.subcore_barrier()
        @pl.when(sid == 0)
        def _():
            pltpu.sync_copy(table, out_hbm.at[q])
        plsc.subcore_barrier()

    for p in range(2):
        for cval in range(2):
            @pl.when(core == cval)
            def _(q=2 * p + cval):
                run_pass(q)


def _sc_scatter_stage(rw, sh, hs, dst, zeros):
    k = pl.kernel(
        _scatter_body,
        out_type=jax.ShapeDtypeStruct((4, N_NODES, 144), jnp.float32),
        mesh=_sc_mesh(),
        scratch_types=[
            pltpu.VMEM_SHARED((N_NODES, 144), jnp.float32),
            pltpu.VMEM((2, S_CHUNK), jnp.int32),
            pltpu.VMEM((2, S_CHUNK), jnp.int32),
            pltpu.VMEM((2, 3, S_CHUNK, 16), jnp.float32),
            pltpu.VMEM((2, S_CHUNK, 16), jnp.float32),
            pltpu.VMEM((2, S_CHUNK, 16), jnp.float32),
            pltpu.VMEM((2, S_CHUNK, 144), jnp.float32),
            pltpu.SemaphoreType.DMA,
            pltpu.SemaphoreType.DMA,
            pltpu.SemaphoreType.DMA,
            pltpu.SemaphoreType.DMA,
        ],
        compiler_params=_sc_params(),
    )
    return k(dst, rw, sh, hs, zeros)


def _sc_scatter(rw, sh, h, src, dst, zeros):
    hs = _sc_hgather_stage(h, src)
    araw = _sc_scatter_stage(rw, sh, hs, dst, zeros)
    return araw


def kernel(coordinates, node_attrs, edge_index, W_embed, W_up, Wr1, Wr2, Wr3,
           W_int, W_mix, Wc, W_msg, W_skip):
    src = edge_index[0].astype(jnp.int32)
    dst = edge_index[1].astype(jnp.int32)
    coords_pad = jnp.pad(coordinates, ((0, 0), (0, 13)))

    ldx = jnp.asarray(np.array([0, 1, 1, 1, 2, 2, 2, 2, 2]))
    W_eff0 = W_embed @ W_up[0]
    Wr3p = Wr3.reshape(2, 64, N_CHAN, 3).transpose(0, 1, 3, 2).reshape(2, 64, 192)
    Wm = W_int[:, ldx] / AVG_NEIGH
    eye9 = jnp.eye(9, dtype=jnp.float32)
    Wq = jnp.einsum('imqcd,mn->iqmcnd', Wm.reshape(2, 9, 4, 16, 64),
                    eye9).reshape(2, 4, 144, 576)
    Wc_p = Wc.reshape(2, 128, N_CHAN, 3).transpose(0, 1, 3, 2).reshape(2, 128, 192)

    vec = _sc_vec_stage(coords_pad, src, dst)
    zeros = jnp.zeros((N_NODES, 144), jnp.float32)

    sh, rw0, rw1 = _edge_stage(vec, Wr1, Wr2, Wr3p)

    h0 = _mm_stage(node_attrs, W_eff0)

    dummy_nf = jnp.zeros((N_NODES, 64), jnp.float32)

    A0 = _sc_scatter(rw0, sh, h0, src, dst, zeros)
    out0, h1 = _node_stage(True, A0, node_attrs, dummy_nf,
                           Wq[0], W_mix, Wc_p[0], W_msg[0], W_skip[0], W_up[1])

    A1 = _sc_scatter(rw1, sh, h1, src, dst, zeros)
    out1, _ = _node_stage(False, A1, node_attrs, out0,
                          Wq[1], W_mix, Wc_p[1], W_msg[1], W_skip[1], W_up[1])

    return jnp.stack([out0, out1], axis=0)
